# Initial kernel scaffold; baseline (speedup 1.0000x reference)
import jax
import jax.numpy as jnp
from jax.experimental import pallas as pl


def kernel(x, W):
    raise NotImplementedError("write your pallas kernel here")


# speedup vs baseline: 1.4583x; 1.4583x over previous
import functools

import jax
import jax.numpy as jnp
from jax import lax
from jax.experimental import pallas as pl
from jax.experimental.pallas import tpu as pltpu
from jax.experimental.pallas import tpu_sc as plsc

_VOCAB = 1000000
_D = 32
_B = 4096
_H = 200
_N = _B * _H
_NW = 32
_PER_W = _N // _NW
_CHUNK = 1024
_NCHUNK = _PER_W // _CHUNK


def _gather_kernel(idx_hbm, w_hbm, out_hbm, idx_v, rows_v, sem):
    wid = lax.axis_index("s") * 2 + lax.axis_index("c")
    base = wid * _PER_W

    def body(g, carry):
        off = base + g * _CHUNK
        pltpu.sync_copy(idx_hbm.at[pl.ds(off, _CHUNK)], idx_v)
        pltpu.async_copy(w_hbm.at[idx_v], rows_v, sem).wait()
        pltpu.sync_copy(rows_v, out_hbm.at[pl.ds(off, _CHUNK)])
        return carry

    lax.fori_loop(0, _NCHUNK, body, 0)


@jax.jit
def _embed(x_flat, W):
    mesh = plsc.VectorSubcoreMesh(core_axis_name="c", subcore_axis_name="s")
    run = pl.kernel(
        _gather_kernel,
        mesh=mesh,
        out_type=jax.ShapeDtypeStruct((_N, _D), jnp.float32),
        scratch_types=[
            pltpu.VMEM((_CHUNK,), jnp.int32),
            pltpu.VMEM((_CHUNK, _D), jnp.float32),
            pltpu.SemaphoreType.DMA,
        ],
        compiler_params=pltpu.CompilerParams(use_tc_tiling_on_sc=False),
    )
    return run(x_flat, W)


def kernel(x, W):
    out = _embed(x.reshape(_N), W)
    return out.reshape(_B, _H, _D)


# speedup vs baseline: 1.5015x; 1.0297x over previous
import jax
import jax.numpy as jnp
from jax import lax
from jax.experimental import pallas as pl
from jax.experimental.pallas import tpu as pltpu
from jax.experimental.pallas import tpu_sc as plsc

_VOCAB = 1000000
_D = 32
_B = 4096
_H = 200
_N = _B * _H
_NW = 32
_PER_W = _N // _NW
_CHUNK = 800
_NCHUNK = _PER_W // _CHUNK
_NBUF = 4
_NOUTER = _NCHUNK // _NBUF


def _gather_kernel(idx_hbm, w_hbm, out_hbm, *refs):
    idx = refs[0:4]
    rows = refs[4:8]
    gsem = refs[8:12]
    wsem = refs[12:16]
    wid = lax.axis_index("s") * 2 + lax.axis_index("c")
    base = wid * _PER_W

    for b in range(_NBUF - 1):
        pltpu.sync_copy(idx_hbm.at[pl.ds(base + b * _CHUNK, _CHUNK)], idx[b])
        pltpu.async_copy(w_hbm.at[idx[b]], rows[b], gsem[b])

    def outer(t, carry):
        for b in range(_NBUF):
            g = t * _NBUF + b
            off = base + g * _CHUNK
            nb = (b + 3) % _NBUF

            def prefetch_idx():
                pltpu.sync_copy(
                    idx_hbm.at[pl.ds(off + 3 * _CHUNK, _CHUNK)], idx[nb]
                )

            def wait_writeback():
                pltpu.make_async_copy(
                    rows[nb], out_hbm.at[pl.ds(base, _CHUNK)], wsem[nb]
                ).wait()

            def start_gather():
                pltpu.async_copy(w_hbm.at[idx[nb]], rows[nb], gsem[nb])

            if b == 0:
                prefetch_idx()
                pl.when(t > 0)(wait_writeback)
                start_gather()
            else:
                def prefetch_all():
                    prefetch_idx()
                    wait_writeback()
                    start_gather()
                pl.when(g + 3 < _NCHUNK)(prefetch_all)

            pltpu.make_async_copy(w_hbm.at[idx[b]], rows[b], gsem[b]).wait()
            pltpu.async_copy(rows[b], out_hbm.at[pl.ds(off, _CHUNK)], wsem[b])
        return carry

    lax.fori_loop(0, _NOUTER, outer, 0)

    for b in range(_NBUF):
        pltpu.make_async_copy(
            rows[b], out_hbm.at[pl.ds(base, _CHUNK)], wsem[b]
        ).wait()


@jax.jit
def _embed(x_flat, W):
    mesh = plsc.VectorSubcoreMesh(core_axis_name="c", subcore_axis_name="s")
    run = pl.kernel(
        _gather_kernel,
        mesh=mesh,
        out_type=jax.ShapeDtypeStruct((_N, _D), jnp.float32),
        scratch_types=(
            [pltpu.VMEM((_CHUNK,), jnp.int32) for _ in range(_NBUF)]
            + [pltpu.VMEM((_CHUNK, _D), jnp.float32) for _ in range(_NBUF)]
            + [pltpu.SemaphoreType.DMA for _ in range(2 * _NBUF)]
        ),
        compiler_params=pltpu.CompilerParams(use_tc_tiling_on_sc=False),
    )
    return run(x_flat, W)


def kernel(x, W):
    out = _embed(x.reshape(_N), W)
    return out.reshape(_B, _H, _D)
